# Initial kernel scaffold; baseline (speedup 1.0000x reference)
import jax
import jax.numpy as jnp
from jax.experimental import pallas as pl


def kernel(x):
    raise NotImplementedError("write your pallas kernel here")


# speedup vs baseline: 65.7179x; 65.7179x over previous
import jax
import jax.numpy as jnp
from jax import lax
from jax.experimental import pallas as pl

_CONF_THRES = 0.001
_IOU_THRES = 0.45
_MAX_DET = 100
_MAX_WH = 4096.0
_N = 5000
_NPAD = 5120
_NC = 80
_B = 4

_NEG_INF = float("-inf")


def _nms_body(pred_ref, out_ref):
    p = pred_ref[...]
    cx = p[:, 0]
    cy = p[:, 1]
    w = p[:, 2]
    h = p[:, 3]
    obj = p[:, 4]
    bx1 = cx - w / 2.0
    by1 = cy - h / 2.0
    bx2 = cx + w / 2.0
    by2 = cy + h / 2.0

    cs = p[:, 5:] * obj[:, None]
    conf = jnp.max(cs, axis=1)
    cls_iota = lax.broadcasted_iota(jnp.int32, (1, _NC, 1, 1), 1)
    j = jnp.min(jnp.where(cs == conf[:, None], cls_iota, _NC), axis=1)
    cls_f = j.astype(jnp.float32)

    off = cls_f * _MAX_WH
    x1 = bx1 + off
    y1 = by1 + off
    x2 = bx2 + off
    y2 = by2 + off
    areas = (x2 - x1) * (y2 - y1)

    s0 = jnp.where(conf > _CONF_THRES, conf, _NEG_INF)

    idx = (128 * lax.broadcasted_iota(jnp.int32, (40, 128), 0)
           + lax.broadcasted_iota(jnp.int32, (40, 128), 1))[None]
    lane100 = lax.broadcasted_iota(jnp.int32, (1, _MAX_DET), 1)

    def extract(mask, v):
        return jnp.sum(jnp.where(mask, v, 0.0), axis=(1, 2))

    def body(k, carry):
        s, ox1, oy1, ox2, oy2, ocf, ocl = carry
        m = jnp.max(s, axis=(1, 2))
        ii = jnp.min(jnp.where(s == m[:, None, None], idx, _NPAD), axis=(1, 2))
        mask1 = idx == ii[:, None, None]

        x1i = extract(mask1, x1)
        y1i = extract(mask1, y1)
        x2i = extract(mask1, x2)
        y2i = extract(mask1, y2)
        bx1i = extract(mask1, bx1)
        by1i = extract(mask1, by1)
        bx2i = extract(mask1, bx2)
        by2i = extract(mask1, by2)
        cli = extract(mask1, cls_f)
        area_i = (x2i - x1i) * (y2i - y1i)

        xx1 = jnp.maximum(x1i[:, None, None], x1)
        yy1 = jnp.maximum(y1i[:, None, None], y1)
        xx2 = jnp.minimum(x2i[:, None, None], x2)
        yy2 = jnp.minimum(y2i[:, None, None], y2)
        inter = jnp.maximum(xx2 - xx1, 0.0) * jnp.maximum(yy2 - yy1, 0.0)
        iou = inter / (area_i[:, None, None] + areas - inter + 1e-9)
        s = jnp.where(iou > _IOU_THRES, _NEG_INF, s)
        s = jnp.where(mask1, _NEG_INF, s)

        keep = m > _CONF_THRES
        sel = (lane100 == k)
        z = 0.0
        ox1 = jnp.where(sel, jnp.where(keep, bx1i, z)[:, None], ox1)
        oy1 = jnp.where(sel, jnp.where(keep, by1i, z)[:, None], oy1)
        ox2 = jnp.where(sel, jnp.where(keep, bx2i, z)[:, None], ox2)
        oy2 = jnp.where(sel, jnp.where(keep, by2i, z)[:, None], oy2)
        ocf = jnp.where(sel, jnp.where(keep, m, z)[:, None], ocf)
        ocl = jnp.where(sel, jnp.where(keep, cli, z)[:, None], ocl)
        return s, ox1, oy1, ox2, oy2, ocf, ocl

    zeros = jnp.zeros((_B, _MAX_DET), jnp.float32)
    carry = (s0, zeros, zeros, zeros, zeros, zeros, zeros)
    _, ox1, oy1, ox2, oy2, ocf, ocl = lax.fori_loop(0, _MAX_DET, body, carry)
    out_ref[...] = jnp.stack([ox1, oy1, ox2, oy2, ocf, ocl], axis=-1)


def kernel(x):
    pred = x[0]
    pt = jnp.transpose(pred, (0, 2, 1))
    pt = jnp.pad(pt, ((0, 0), (0, 0), (0, _NPAD - _N)))
    pt = pt.reshape(_B, 85, 40, 128)
    return pl.pallas_call(
        _nms_body,
        out_shape=jax.ShapeDtypeStruct((_B, _MAX_DET, 6), jnp.float32),
    )(pt)
